# baseline (speedup 1.0000x reference)
import functools

import jax
import jax.numpy as jnp
from jax import lax
from jax.experimental import pallas as pl
from jax.experimental.pallas import tpu as pltpu
from jax.experimental.pallas import tpu_sc as plsc

D_MODEL = 768
NUM_WORKERS = 32
CHUNK = 32
LANES = 16


def _build_sc_call(n_chunks, table_rows):
    mesh = plsc.VectorSubcoreMesh(core_axis_name="c", subcore_axis_name="s")

    @functools.partial(
        pl.kernel,
        out_type=jax.ShapeDtypeStruct(
            (NUM_WORKERS, n_chunks, CHUNK, D_MODEL), jnp.float32
        ),
        mesh=mesh,
        scratch_types=[
            pltpu.VMEM((n_chunks, CHUNK), jnp.int32),
            pltpu.VMEM((CHUNK, D_MODEL), jnp.float32),
            pltpu.VMEM((CHUNK, D_MODEL), jnp.float32),
            pltpu.SemaphoreType.DMA,
            pltpu.SemaphoreType.DMA,
            pltpu.SemaphoreType.DMA,
        ],
    )
    def sc_call(x_hbm, pos_hbm, tab_hbm, out_hbm, idx_v, rows_v, xb_v,
                gsem, xsem, osem):
        wid = lax.axis_index("s") * 2 + lax.axis_index("c")
        pltpu.sync_copy(pos_hbm.at[wid], idx_v)

        def chunk_body(c, _):
            g = pltpu.async_copy(tab_hbm.at[idx_v.at[c]], rows_v, gsem)
            xcp = pltpu.async_copy(x_hbm.at[wid, c], xb_v, xsem)
            g.wait()
            xcp.wait()

            def tok_body(t, _):
                for d in range(D_MODEL // LANES):
                    sl = pl.ds(d * LANES, LANES)
                    plsc.addupdate(xb_v.at[t, sl], rows_v[t, sl])
                return 0

            lax.fori_loop(0, CHUNK, tok_body, 0)
            pltpu.async_copy(xb_v, out_hbm.at[wid, c], osem).wait()
            return 0

        lax.fori_loop(0, n_chunks, chunk_body, 0)

    return sc_call


def kernel(x, positions, position_embeddings):
    b, s, d = x.shape
    assert d == D_MODEL
    total = b * s
    tokens_per_worker = total // NUM_WORKERS
    n_chunks = tokens_per_worker // CHUNK

    xf = x.reshape(NUM_WORKERS, n_chunks, CHUNK, d)
    posf = positions.reshape(NUM_WORKERS, n_chunks, CHUNK).astype(jnp.int32)

    sc_call = _build_sc_call(n_chunks, position_embeddings.shape[0])
    out = sc_call(xf, posf, position_embeddings)
    return out.reshape(b, s, d)


# speedup vs baseline: 1.1412x; 1.1412x over previous
import functools

import jax
import jax.numpy as jnp
from jax import lax
from jax.experimental import pallas as pl
from jax.experimental.pallas import tpu as pltpu
from jax.experimental.pallas import tpu_sc as plsc

D_MODEL = 768
NUM_WORKERS = 32
CHUNK = 32
LANES = 16


def _build_sc_call(n_chunks):
    mesh = plsc.VectorSubcoreMesh(core_axis_name="c", subcore_axis_name="s")
    n_half = n_chunks // 2

    @functools.partial(
        pl.kernel,
        out_type=jax.ShapeDtypeStruct(
            (NUM_WORKERS, n_chunks, CHUNK, D_MODEL), jnp.float32
        ),
        mesh=mesh,
        scratch_types=[
            pltpu.VMEM((n_chunks, CHUNK), jnp.int32),
            pltpu.VMEM((2, CHUNK, D_MODEL), jnp.float32),
            pltpu.VMEM((2, CHUNK, D_MODEL), jnp.float32),
            pltpu.SemaphoreType.DMA,
            pltpu.SemaphoreType.DMA,
            pltpu.SemaphoreType.DMA,
            pltpu.SemaphoreType.DMA,
            pltpu.SemaphoreType.DMA,
            pltpu.SemaphoreType.DMA,
        ],
    )
    def sc_call(x_hbm, pos_hbm, tab_hbm, out_hbm, idx_v, rows_v, xb_v,
                gsem0, gsem1, xsem0, xsem1, osem0, osem1):
        gsem = (gsem0, gsem1)
        xsem = (xsem0, xsem1)
        osem = (osem0, osem1)
        wid = lax.axis_index("s") * 2 + lax.axis_index("c")
        pltpu.sync_copy(pos_hbm.at[wid], idx_v)

        def start_gather(slot, c):
            pltpu.async_copy(tab_hbm.at[idx_v.at[c]], rows_v.at[slot],
                             gsem[slot])

        def start_xload(slot, c):
            pltpu.async_copy(x_hbm.at[wid, c], xb_v.at[slot], xsem[slot])

        def start_store(slot, c):
            pltpu.async_copy(xb_v.at[slot], out_hbm.at[wid, c], osem[slot])

        def wait_load(slot, c):
            pltpu.make_async_copy(tab_hbm.at[idx_v.at[c]], rows_v.at[slot],
                                  gsem[slot]).wait()
            pltpu.make_async_copy(x_hbm.at[wid, c], xb_v.at[slot],
                                  xsem[slot]).wait()

        def wait_store(slot, c):
            pltpu.make_async_copy(xb_v.at[slot], out_hbm.at[wid, c],
                                  osem[slot]).wait()

        def accumulate(slot):
            def tok_body(t, _):
                for d in range(D_MODEL // LANES):
                    sl = pl.ds(d * LANES, LANES)
                    plsc.addupdate(xb_v.at[slot, t, sl], rows_v[slot, t, sl])
                return 0

            lax.fori_loop(0, CHUNK, tok_body, 0)

        for slot in (0, 1):
            start_gather(slot, slot)
            start_xload(slot, slot)

        def pipe_body(i, _):
            c0 = 2 * i
            for slot in (0, 1):
                c = c0 + slot
                wait_load(slot, c)
                accumulate(slot)
                start_store(slot, c)
                start_gather(slot, c + 2)
                wait_store(slot, c)
                start_xload(slot, c + 2)
            return 0

        lax.fori_loop(0, n_half - 1, pipe_body, 0)

        for slot in (0, 1):
            c = n_chunks - 2 + slot
            wait_load(slot, c)
            accumulate(slot)
            start_store(slot, c)
        for slot in (0, 1):
            wait_store(slot, n_chunks - 2 + slot)

    return sc_call


def kernel(x, positions, position_embeddings):
    b, s, d = x.shape
    assert d == D_MODEL
    total = b * s
    tokens_per_worker = total // NUM_WORKERS
    n_chunks = tokens_per_worker // CHUNK

    xf = x.reshape(NUM_WORKERS, n_chunks, CHUNK, d)
    posf = positions.reshape(NUM_WORKERS, n_chunks, CHUNK).astype(jnp.int32)

    sc_call = _build_sc_call(n_chunks)
    out = sc_call(xf, posf, position_embeddings)
    return out.reshape(b, s, d)
